# Initial kernel scaffold; baseline (speedup 1.0000x reference)
import jax
import jax.numpy as jnp
from jax.experimental import pallas as pl


def kernel(src_table, tgt_table, src_indices, tgt_indices):
    raise NotImplementedError("write your pallas kernel here")


# speedup vs baseline: 1.1283x; 1.1283x over previous
import jax
import jax.numpy as jnp
from jax import lax
from jax.experimental import pallas as pl
from jax.experimental.pallas import tpu as pltpu
from jax.experimental.pallas import tpu_sc as plsc

NC = 2
NS = 16
L = 16
NW = NC * NS

D = 32
B_TOT = 16384 * 50
PER_W = B_TOT // NW
C = 1280
SUB = 128
NSUB = C // SUB
GPC = C // L
NCH = PER_W // C


def _table_pass(table, idxh, outh, idx_v, rows_v, semg, semw, base):
    def one_chunk(k, b, p):
        start = base + k * C

        @pl.when(p > 0)
        def _w():
            pltpu.make_async_copy(
                rows_v.at[b], outh.at[pl.ds(start - 2 * C, C)], semw.at[b]
            ).wait()

        pltpu.sync_copy(idxh.at[pl.ds(start, C)], idx_v.at[b])

        handles = [
            pltpu.async_copy(
                table.at[idx_v.at[b].at[pl.ds(j * SUB, SUB)]],
                rows_v.at[b].at[pl.ds(j * SUB, SUB)],
                semg,
            )
            for j in range(NSUB)
        ]
        for h in handles:
            h.wait()

        def scan_min(g, acc):
            return jnp.minimum(acc, idx_v[b, pl.ds(g * L, L)])

        acc = lax.fori_loop(
            0, GPC, scan_min, jnp.full((L,), 2**31 - 1, jnp.int32)
        )
        m0 = acc[0]
        for i in range(1, L):
            m0 = jnp.minimum(m0, acc[i])

        @pl.when(m0 == 0)
        def _fix():
            def fixg(g, carry):
                vec = idx_v[b, pl.ds(g * L, L)]
                for i in range(L):
                    scal = jnp.where(vec[i] == 0, 0.0, 1.0).astype(jnp.float32)
                    r = g * L + i
                    for h in range(D // L):
                        sl = pl.ds(h * L, L)
                        rows_v[b, r, sl] = rows_v[b, r, sl] * scal
                return carry

            lax.fori_loop(0, GPC, fixg, 0)

        pltpu.async_copy(rows_v.at[b], outh.at[pl.ds(start, C)], semw.at[b])

    def pair(p, carry):
        for b in range(2):
            one_chunk(2 * p + b, b, p)
        return carry

    lax.fori_loop(0, NCH // 2, pair, 0)

    for b in range(2):
        k = NCH - 2 + b
        pltpu.make_async_copy(
            rows_v.at[b], outh.at[pl.ds(base + k * C, C)], semw.at[b]
        ).wait()


def _body(src_table, tgt_table, src_idx, tgt_idx, src_out, tgt_out,
          idx_v, rows_v, semg, semw):
    wid = lax.axis_index("s") * NC + lax.axis_index("c")
    base = wid * PER_W
    _table_pass(src_table, src_idx, src_out, idx_v, rows_v, semg, semw, base)
    _table_pass(tgt_table, tgt_idx, tgt_out, idx_v, rows_v, semg, semw, base)


def kernel(src_table, tgt_table, src_indices, tgt_indices):
    out_shape = src_indices.shape + (D,)
    src_flat = src_indices.reshape(-1).astype(jnp.int32)
    tgt_flat = tgt_indices.reshape(-1).astype(jnp.int32)

    mesh = plsc.VectorSubcoreMesh(core_axis_name="c", subcore_axis_name="s")
    k = pl.kernel(
        _body,
        out_type=(
            jax.ShapeDtypeStruct((B_TOT, D), jnp.float32),
            jax.ShapeDtypeStruct((B_TOT, D), jnp.float32),
        ),
        mesh=mesh,
        compiler_params=pltpu.CompilerParams(use_tc_tiling_on_sc=False),
        scratch_types=[
            pltpu.VMEM((2, C), jnp.int32),
            pltpu.VMEM((2, C, D), jnp.float32),
            pltpu.SemaphoreType.DMA,
            pltpu.SemaphoreType.DMA((2,)),
        ],
    )
    src_out, tgt_out = k(src_table, tgt_table, src_flat, tgt_flat)
    return (src_out.reshape(out_shape), tgt_out.reshape(out_shape))


# speedup vs baseline: 1.9269x; 1.7078x over previous
import jax
import jax.numpy as jnp
from jax import lax
from jax.experimental import pallas as pl
from jax.experimental.pallas import tpu as pltpu
from jax.experimental.pallas import tpu_sc as plsc

NC = 2
NS = 16
L = 16
NW = NC * NS

D = 32
NB = 16384
NL = 50
C = NB // NW
SUB = 128
NSUB = C // SUB
GPC = C // L


def _table_pass(table, idxh, outh, idx_all, rows_v, semg, semw, b0):
    pltpu.sync_copy(idxh.at[:, pl.ds(b0, C)], idx_all)

    def fire_gathers(k, b):
        for j in range(NSUB):
            pltpu.async_copy(
                table.at[idx_all.at[k, pl.ds(j * SUB, SUB)]],
                rows_v.at[b, pl.ds(j * SUB, SUB)],
                semg.at[b],
            )

    def drain_gathers(k, b):
        for j in range(NSUB):
            pltpu.make_async_copy(
                table.at[idx_all.at[k, pl.ds(j * SUB, SUB)]],
                rows_v.at[b, pl.ds(j * SUB, SUB)],
                semg.at[b],
            ).wait()

    def one_chunk(k, b, p):
        nb = 1 - b

        @pl.when(k + 1 < NL)
        def _g():
            @pl.when(k >= 1)
            def _w():
                pltpu.make_async_copy(
                    rows_v.at[nb],
                    outh.at[k - 1, pl.ds(b0, C), :],
                    semw.at[nb],
                ).wait()

            fire_gathers(k + 1, nb)

        drain_gathers(k, b)

        def scan_min(g, acc):
            return jnp.minimum(acc, idx_all[k, pl.ds(g * L, L)])

        acc = lax.fori_loop(
            0, GPC, scan_min, jnp.full((L,), 2**31 - 1, jnp.int32)
        )
        m0 = acc[0]
        for i in range(1, L):
            m0 = jnp.minimum(m0, acc[i])

        @pl.when(m0 == 0)
        def _fix():
            def fixg(g, carry):
                vec = idx_all[k, pl.ds(g * L, L)]
                for i in range(L):
                    scal = jnp.where(vec[i] == 0, 0.0, 1.0).astype(jnp.float32)
                    r = g * L + i
                    for h in range(D // L):
                        sl = pl.ds(h * L, L)
                        rows_v[b, r, sl] = rows_v[b, r, sl] * scal
                return carry

            lax.fori_loop(0, GPC, fixg, 0)

        pltpu.async_copy(
            rows_v.at[b], outh.at[k, pl.ds(b0, C), :], semw.at[b]
        )

    fire_gathers(0, 0)

    def pair(p, carry):
        for b in range(2):
            one_chunk(2 * p + b, b, p)
        return carry

    lax.fori_loop(0, NL // 2, pair, 0)

    for b in range(2):
        k = NL - 2 + b
        pltpu.make_async_copy(
            rows_v.at[b], outh.at[k, pl.ds(b0, C), :], semw.at[b]
        ).wait()


def _body(src_table, tgt_table, src_idxT, tgt_idxT, src_out, tgt_out,
          idx_all, rows_v, semg, semw):
    wid = lax.axis_index("s") * NC + lax.axis_index("c")
    b0 = wid * C
    _table_pass(src_table, src_idxT, src_out,
                idx_all, rows_v, semg, semw, b0)
    _table_pass(tgt_table, tgt_idxT, tgt_out,
                idx_all, rows_v, semg, semw, b0)


def kernel(src_table, tgt_table, src_indices, tgt_indices):
    mesh = plsc.VectorSubcoreMesh(core_axis_name="c", subcore_axis_name="s")
    k = pl.kernel(
        _body,
        out_type=(
            jax.ShapeDtypeStruct((NL, NB, D), jnp.float32),
            jax.ShapeDtypeStruct((NL, NB, D), jnp.float32),
        ),
        mesh=mesh,
        compiler_params=pltpu.CompilerParams(use_tc_tiling_on_sc=False),
        scratch_types=[
            pltpu.VMEM((NL, C), jnp.int32),
            pltpu.VMEM((2, C, D), jnp.float32),
            pltpu.SemaphoreType.DMA((2,)),
            pltpu.SemaphoreType.DMA((2,)),
        ],
    )
    src_mid, tgt_mid = k(
        src_table, tgt_table,
        src_indices.astype(jnp.int32).T, tgt_indices.astype(jnp.int32).T,
    )
    return (src_mid.transpose(1, 0, 2), tgt_mid.transpose(1, 0, 2))


# speedup vs baseline: 1.9851x; 1.0302x over previous
import jax
import jax.numpy as jnp
from jax import lax
from jax.experimental import pallas as pl
from jax.experimental.pallas import tpu as pltpu
from jax.experimental.pallas import tpu_sc as plsc

NC = 2
NS = 16
L = 16
NW = NC * NS

D = 32
NB = 16384
NL = 50
C = NB // NW
SUB = 128
NSUB = C // SUB
GPC = C // L


def _table_pass(table, idxh, outh, idx_all, rows_v, semg, semw, b0):
    pltpu.sync_copy(idxh.at[:, pl.ds(b0, C)], idx_all)

    def fire_gathers(k, b):
        for j in range(NSUB):
            pltpu.async_copy(
                table.at[idx_all.at[k, pl.ds(j * SUB, SUB)]],
                rows_v.at[b, pl.ds(j * SUB, SUB)],
                semg.at[b],
            )

    def drain_gathers(k, b):
        for j in range(NSUB):
            pltpu.make_async_copy(
                table.at[idx_all.at[k, pl.ds(j * SUB, SUB)]],
                rows_v.at[b, pl.ds(j * SUB, SUB)],
                semg.at[b],
            ).wait()

    def one_chunk(k, b, p):
        nb = 1 - b

        @pl.when(k + 1 < NL)
        def _g():
            @pl.when(k >= 1)
            def _w():
                pltpu.make_async_copy(
                    rows_v.at[nb],
                    outh.at[k - 1, pl.ds(b0, C), :],
                    semw.at[nb],
                ).wait()

            fire_gathers(k + 1, nb)

        drain_gathers(k, b)

        def scan_min(g, acc):
            return jnp.minimum(acc, idx_all[k, pl.ds(g * L, L)])

        acc = lax.fori_loop(
            0, GPC, scan_min, jnp.full((L,), 2**31 - 1, jnp.int32)
        )
        m0 = acc[0]
        for i in range(1, L):
            m0 = jnp.minimum(m0, acc[i])

        @pl.when(m0 == 0)
        def _fix():
            def fixg(g, carry):
                vec = idx_all[k, pl.ds(g * L, L)]
                for i in range(L):
                    scal = jnp.where(vec[i] == 0, 0.0, 1.0).astype(jnp.float32)
                    r = g * L + i
                    for h in range(D // L):
                        sl = pl.ds(h * L, L)
                        rows_v[b, r, sl] = rows_v[b, r, sl] * scal
                return carry

            lax.fori_loop(0, GPC, fixg, 0)

        pltpu.async_copy(
            rows_v.at[b], outh.at[k, pl.ds(b0, C), :], semw.at[b]
        )

    fire_gathers(0, 0)

    def pair(p, carry):
        for b in range(2):
            one_chunk(2 * p + b, b, p)
        return carry

    lax.fori_loop(0, NL // 2, pair, 0)

    for b in range(2):
        k = NL - 2 + b
        pltpu.make_async_copy(
            rows_v.at[b], outh.at[k, pl.ds(b0, C), :], semw.at[b]
        ).wait()


def _body(table, idxT, out, idx_all, rows_v, semg, semw):
    wid = lax.axis_index("s") * NC + lax.axis_index("c")
    b0 = wid * C
    _table_pass(table, idxT, out, idx_all, rows_v, semg, semw, b0)


def kernel(src_table, tgt_table, src_indices, tgt_indices):
    mesh = plsc.VectorSubcoreMesh(core_axis_name="c", subcore_axis_name="s")
    k = pl.kernel(
        _body,
        out_type=jax.ShapeDtypeStruct((NL, NB, D), jnp.float32),
        mesh=mesh,
        compiler_params=pltpu.CompilerParams(use_tc_tiling_on_sc=False),
        scratch_types=[
            pltpu.VMEM((NL, C), jnp.int32),
            pltpu.VMEM((2, C, D), jnp.float32),
            pltpu.SemaphoreType.DMA((2,)),
            pltpu.SemaphoreType.DMA((2,)),
        ],
    )
    src_mid = k(src_table, src_indices.astype(jnp.int32).T)
    tgt_mid = k(tgt_table, tgt_indices.astype(jnp.int32).T)
    return (src_mid.transpose(1, 0, 2), tgt_mid.transpose(1, 0, 2))
